# baseline (speedup 1.0000x reference)
import jax
import jax.numpy as jnp
from jax.experimental import pallas as pl
from jax.experimental.pallas import tpu as pltpu

N = 65536
D = 32
BLK = 8192
NB = N // BLK
BETA = 0.25


def _argmin_onehot(dt):
    dmin = jnp.min(dt, axis=0, keepdims=True)
    iota_f = jax.lax.broadcasted_iota(jnp.int32, dt.shape, 0).astype(jnp.float32)
    masked = jnp.where(dt == dmin, iota_f, 3.0e38)
    cmin = jnp.min(masked, axis=0, keepdims=True)
    onehot = iota_f == cmin
    return cmin.astype(jnp.int32), onehot


def _gather_rows(cb, onehot_bf):
    hi = cb.astype(jnp.bfloat16)
    rem1 = cb - hi.astype(jnp.float32)
    mid = rem1.astype(jnp.bfloat16)
    lo = (rem1 - mid.astype(jnp.float32)).astype(jnp.bfloat16)
    dot = lambda a: jax.lax.dot_general(
        a, onehot_bf, (((0,), (0,)), ((), ())),
        preferred_element_type=jnp.float32)
    return (dot(hi) + dot(mid)) + dot(lo)


def _level(rt, cb):
    csq = jnp.sum(cb * cb, axis=1, keepdims=True)
    st = jax.lax.dot_general(cb.astype(jnp.bfloat16), rt.astype(jnp.bfloat16),
                             (((1,), (0,)), ((), ())),
                             preferred_element_type=jnp.float32)
    dt = csq - 2.0 * st
    codes, onehot = _argmin_onehot(dt)
    et = _gather_rows(cb, onehot.astype(jnp.bfloat16))
    return codes, et


def _rvq_body(z_ref, cb0_ref, cb1_ref, cb2_ref,
              c0_ref, c1_ref, c2_ref, q_ref, loss_ref):
    zt = z_ref[...].T
    c0, e0 = _level(zt, cb0_ref[...])
    r1 = zt - e0
    c1, e1 = _level(r1, cb1_ref[...])
    r2 = r1 - e1
    c2, e2 = _level(r2, cb2_ref[...])
    r3 = r2 - e2

    c0_ref[...] = c0.reshape(1, 1, BLK)
    c1_ref[...] = c1.reshape(1, 1, BLK)
    c2_ref[...] = c2.reshape(1, 1, BLK)
    q_ref[...] = ((e0 + e1) + e2).T

    part = (jnp.sum(r1 * r1) + jnp.sum(r2 * r2) + jnp.sum(r3 * r3))
    part = jnp.reshape(part * ((1.0 + BETA) / (N * D)), (1, 1))
    i = pl.program_id(0)

    @pl.when(i == 0)
    def _init():
        loss_ref[...] = part

    @pl.when(i != 0)
    def _acc():
        loss_ref[...] += part


def kernel(z, cb0, cb1, cb2):
    out_shape = (
        jax.ShapeDtypeStruct((NB, 1, BLK), jnp.int32),
        jax.ShapeDtypeStruct((NB, 1, BLK), jnp.int32),
        jax.ShapeDtypeStruct((NB, 1, BLK), jnp.int32),
        jax.ShapeDtypeStruct((N, D), jnp.float32),
        jax.ShapeDtypeStruct((1, 1), jnp.float32),
    )
    full = lambda shape: pl.BlockSpec(shape, lambda i: tuple(0 for _ in shape))
    c0, c1, c2, q, loss = pl.pallas_call(
        _rvq_body,
        grid=(NB,),
        in_specs=[
            pl.BlockSpec((BLK, D), lambda i: (i, 0)),
            full(cb0.shape),
            full(cb1.shape),
            full(cb2.shape),
        ],
        out_specs=(
            pl.BlockSpec((1, 1, BLK), lambda i: (i, 0, 0)),
            pl.BlockSpec((1, 1, BLK), lambda i: (i, 0, 0)),
            pl.BlockSpec((1, 1, BLK), lambda i: (i, 0, 0)),
            pl.BlockSpec((BLK, D), lambda i: (i, 0)),
            pl.BlockSpec((1, 1), lambda i: (0, 0)),
        ),
        out_shape=out_shape,
        compiler_params=pltpu.CompilerParams(
            dimension_semantics=("arbitrary",)),
    )(z, cb0, cb1, cb2)
    return (c0.reshape(N), c1.reshape(N), c2.reshape(N), q, loss[0, 0])


# speedup vs baseline: 1.8931x; 1.8931x over previous
import jax
import jax.numpy as jnp
from jax.experimental import pallas as pl
from jax.experimental.pallas import tpu as pltpu

N = 65536
D = 32
BLK = 16384
NB = N // BLK
BETA = 0.25


def _argmin_onehot(dt):
    dmin = jnp.min(dt, axis=0, keepdims=True)
    iota_f = jax.lax.broadcasted_iota(jnp.int32, dt.shape, 0).astype(jnp.float32)
    masked = jnp.where(dt == dmin, iota_f, 3.0e38)
    cmin = jnp.min(masked, axis=0, keepdims=True)
    onehot = iota_f == cmin
    return cmin.astype(jnp.int32), onehot


def _gather_rows(cb, onehot_bf):
    hi = cb.astype(jnp.bfloat16)
    rem1 = cb - hi.astype(jnp.float32)
    mid = rem1.astype(jnp.bfloat16)
    lo = (rem1 - mid.astype(jnp.float32)).astype(jnp.bfloat16)
    dot = lambda a: jax.lax.dot_general(
        a, onehot_bf, (((0,), (0,)), ((), ())),
        preferred_element_type=jnp.float32)
    return (dot(hi) + dot(mid)) + dot(lo)


def _level(rt, cb):
    csq = jnp.sum(cb * cb, axis=1, keepdims=True)
    st = jax.lax.dot_general(cb.astype(jnp.bfloat16), rt.astype(jnp.bfloat16),
                             (((1,), (0,)), ((), ())),
                             preferred_element_type=jnp.float32)
    dt = csq - 2.0 * st
    codes, onehot = _argmin_onehot(dt)
    et = _gather_rows(cb, onehot.astype(jnp.bfloat16))
    return codes, et


def _rvq_body(zt_ref, cb0_ref, cb1_ref, cb2_ref,
              c0_ref, c1_ref, c2_ref, qt_ref, loss_ref):
    zt = zt_ref[...]
    c0, e0 = _level(zt, cb0_ref[...])
    r1 = zt - e0
    c1, e1 = _level(r1, cb1_ref[...])
    r2 = r1 - e1
    c2, e2 = _level(r2, cb2_ref[...])
    r3 = r2 - e2

    c0_ref[...] = c0.reshape(1, 1, BLK)
    c1_ref[...] = c1.reshape(1, 1, BLK)
    c2_ref[...] = c2.reshape(1, 1, BLK)
    qt_ref[...] = (e0 + e1) + e2

    part = (jnp.sum(r1 * r1) + jnp.sum(r2 * r2) + jnp.sum(r3 * r3))
    part = jnp.reshape(part * ((1.0 + BETA) / (N * D)), (1, 1))
    i = pl.program_id(0)

    @pl.when(i == 0)
    def _init():
        loss_ref[...] = part

    @pl.when(i != 0)
    def _acc():
        loss_ref[...] += part


def kernel(z, cb0, cb1, cb2):
    zt = z.T
    out_shape = (
        jax.ShapeDtypeStruct((NB, 1, BLK), jnp.int32),
        jax.ShapeDtypeStruct((NB, 1, BLK), jnp.int32),
        jax.ShapeDtypeStruct((NB, 1, BLK), jnp.int32),
        jax.ShapeDtypeStruct((D, N), jnp.float32),
        jax.ShapeDtypeStruct((1, 1), jnp.float32),
    )
    full = lambda shape: pl.BlockSpec(shape, lambda i: tuple(0 for _ in shape))
    c0, c1, c2, qt, loss = pl.pallas_call(
        _rvq_body,
        grid=(NB,),
        in_specs=[
            pl.BlockSpec((D, BLK), lambda i: (0, i)),
            full(cb0.shape),
            full(cb1.shape),
            full(cb2.shape),
        ],
        out_specs=(
            pl.BlockSpec((1, 1, BLK), lambda i: (i, 0, 0)),
            pl.BlockSpec((1, 1, BLK), lambda i: (i, 0, 0)),
            pl.BlockSpec((1, 1, BLK), lambda i: (i, 0, 0)),
            pl.BlockSpec((D, BLK), lambda i: (0, i)),
            pl.BlockSpec((1, 1), lambda i: (0, 0)),
        ),
        out_shape=out_shape,
        compiler_params=pltpu.CompilerParams(
            dimension_semantics=("arbitrary",)),
    )(zt, cb0, cb1, cb2)
    return (c0.reshape(N), c1.reshape(N), c2.reshape(N), qt.T, loss[0, 0])
